# baseline (speedup 1.0000x reference)
import jax
import jax.numpy as jnp
from jax.experimental import pallas as pl
from jax.experimental.pallas import tpu as pltpu

NUM_EXPERTS = 64
EXPERT_CAPACITY = 160
BT = 1024
NK = 2


def _router_kernel(hs_ref, w_ref, idx_ref, pmax_ref, logits_ref,
                   acc_ref, counts_ref):
    t = pl.program_id(1)
    k = pl.program_id(2)

    @pl.when(t == 0)
    def _reset():
        counts_ref[...] = jnp.zeros_like(counts_ref)

    x = hs_ref[0]
    part = jax.lax.dot_general(
        w_ref[...], x, (((1,), (1,)), ((), ())),
        preferred_element_type=jnp.float32)

    @pl.when(k < NK - 1)
    def _accum():
        acc_ref[...] = jnp.where(k == 0, part, acc_ref[...] + part)

    @pl.when(k == NK - 1)
    def _epilogue():
        logits_t = acc_ref[...] + part

        m = jnp.max(logits_t, axis=0, keepdims=True)
        z = jnp.sum(jnp.exp(logits_t - m), axis=0, keepdims=True)

        iota = jax.lax.broadcasted_iota(jnp.int32, logits_t.shape, 0)
        cand = jnp.where(logits_t == m, iota, NUM_EXPERTS)
        amin = jnp.min(cand, axis=0, keepdims=True)
        oh = (iota == amin).astype(jnp.float32)

        row = jax.lax.broadcasted_iota(jnp.int32, (BT, BT), 0)
        col = jax.lax.broadcasted_iota(jnp.int32, (BT, BT), 1)
        tri = (row <= col).astype(jnp.float32)
        prio = jnp.dot(oh, tri, preferred_element_type=jnp.float32)
        prio = prio + counts_ref[...]
        counts_ref[...] = prio[:, BT - 1:BT]

        keep = prio <= float(EXPERT_CAPACITY)
        idx_ref[0] = jnp.where(keep, oh, 0.0).astype(jnp.int32)
        pmax_ref[0] = 1.0 / z
        logits_ref[0] = logits_t


@jax.jit
def kernel(hidden_states, W):
    G, T, H = hidden_states.shape
    E = W.shape[0]
    grid = (G, T // BT, NK)
    idx_t, pmax, logits_t = pl.pallas_call(
        _router_kernel,
        grid=grid,
        in_specs=[
            pl.BlockSpec((1, BT, H // NK), lambda g, t, k: (g, t, k)),
            pl.BlockSpec((E, H // NK), lambda g, t, k: (0, k)),
        ],
        out_specs=[
            pl.BlockSpec((1, E, BT), lambda g, t, k: (g, 0, t)),
            pl.BlockSpec((1, 1, BT), lambda g, t, k: (g, 0, t)),
            pl.BlockSpec((1, E, BT), lambda g, t, k: (g, 0, t)),
        ],
        out_shape=[
            jax.ShapeDtypeStruct((G, E, T), jnp.int32),
            jax.ShapeDtypeStruct((G, 1, T), jnp.float32),
            jax.ShapeDtypeStruct((G, E, T), jnp.float32),
        ],
        scratch_shapes=[
            pltpu.VMEM((E, BT), jnp.float32),
            pltpu.VMEM((E, 1), jnp.float32),
        ],
        compiler_params=pltpu.CompilerParams(
            dimension_semantics=("parallel", "arbitrary", "arbitrary")),
    )(hidden_states, W)
    expert_index = jnp.transpose(idx_t, (0, 2, 1))
    router_probs_max = jnp.transpose(pmax, (0, 2, 1))
    router_logits = jnp.transpose(logits_t, (0, 2, 1))
    return expert_index, router_probs_max, router_logits


# speedup vs baseline: 1.0926x; 1.0926x over previous
import jax
import jax.numpy as jnp
from jax.experimental import pallas as pl
from jax.experimental.pallas import tpu as pltpu

NUM_EXPERTS = 64
EXPERT_CAPACITY = 160
BT = 1024


def _router_kernel(hs_ref, w_ref, idx_ref, pmax_ref, logits_ref, counts_ref):
    t = pl.program_id(1)

    @pl.when(t == 0)
    def _reset():
        counts_ref[...] = jnp.zeros_like(counts_ref)

    x = hs_ref[0]
    logits_t = jax.lax.dot_general(
        w_ref[...], x, (((1,), (1,)), ((), ())),
        preferred_element_type=jnp.float32)

    m = jnp.max(logits_t, axis=0, keepdims=True)
    z = jnp.sum(jnp.exp(logits_t - m), axis=0, keepdims=True)

    iota = jax.lax.broadcasted_iota(jnp.int32, logits_t.shape, 0)
    cand = jnp.where(logits_t == m, iota, NUM_EXPERTS)
    amin = jnp.min(cand, axis=0, keepdims=True)
    oh = (iota == amin).astype(jnp.float32)

    row = jax.lax.broadcasted_iota(jnp.int32, (BT, BT), 0)
    col = jax.lax.broadcasted_iota(jnp.int32, (BT, BT), 1)
    tri = (row <= col).astype(jnp.float32)
    prio = jnp.dot(oh, tri, preferred_element_type=jnp.float32)
    prio = prio + counts_ref[...]
    counts_ref[...] = prio[:, BT - 1:BT]

    keep = prio <= float(EXPERT_CAPACITY)
    idx_ref[0] = jnp.where(keep, oh, 0.0).astype(jnp.int32)
    pmax_ref[0] = 1.0 / z
    logits_ref[0] = logits_t


@jax.jit
def kernel(hidden_states, W):
    G, T, H = hidden_states.shape
    E = W.shape[0]
    grid = (G, T // BT)
    idx_t, pmax, logits_t = pl.pallas_call(
        _router_kernel,
        grid=grid,
        in_specs=[
            pl.BlockSpec((1, BT, H), lambda g, t: (g, t, 0)),
            pl.BlockSpec((E, H), lambda g, t: (0, 0)),
        ],
        out_specs=[
            pl.BlockSpec((1, E, BT), lambda g, t: (g, 0, t)),
            pl.BlockSpec((1, 1, BT), lambda g, t: (g, 0, t)),
            pl.BlockSpec((1, E, BT), lambda g, t: (g, 0, t)),
        ],
        out_shape=[
            jax.ShapeDtypeStruct((G, E, T), jnp.int32),
            jax.ShapeDtypeStruct((G, 1, T), jnp.float32),
            jax.ShapeDtypeStruct((G, E, T), jnp.float32),
        ],
        scratch_shapes=[pltpu.VMEM((E, 1), jnp.float32)],
        compiler_params=pltpu.CompilerParams(
            dimension_semantics=("parallel", "arbitrary")),
    )(hidden_states, W)
    expert_index = jnp.transpose(idx_t, (0, 2, 1))
    router_probs_max = jnp.transpose(pmax, (0, 2, 1))
    router_logits = jnp.transpose(logits_t, (0, 2, 1))
    return expert_index, router_probs_max, router_logits
